# baseline (speedup 1.0000x reference)
import jax
import jax.numpy as jnp
from jax import lax
from jax.experimental import pallas as pl
from jax.experimental.pallas import tpu as pltpu
from jax.experimental.pallas import tpu_sc as plsc

N = 10000
E = 320000
D = 128
N_PAD = 10240
ACC_ROWS = 10112
D_AUG = 144
NW = 32
K = 64
C = 6 * (-(-E // (NW * K * 6)))
E_PAD = NW * C * K
STRIPE = ACC_ROWS // 16
BLK = 512


def _make_agg(d, nbuf):
    mesh = plsc.VectorSubcoreMesh(core_axis_name="c", subcore_axis_name="s")

    def body(x_hbm, src_hbm, dst_hbm, z_hbm, out_hbm, src_v, dst_v,
             bufs, sems, acc):
        c = lax.axis_index("c")
        s = lax.axis_index("s")
        wid = c * 16 + s
        pltpu.sync_copy(src_hbm.at[wid], src_v)
        pltpu.sync_copy(dst_hbm.at[wid], dst_v)
        pltpu.sync_copy(z_hbm, acc.at[pl.ds(s * STRIPE, STRIPE)])
        plsc.subcore_barrier()

        for b in range(nbuf):
            pltpu.async_copy(x_hbm.at[src_v.at[b]], bufs[b], sems[b])

        def group(g, carry):
            for b in range(nbuf):
                j = g * nbuf + b
                pltpu.make_async_copy(x_hbm.at[src_v.at[j]], bufs[b],
                                      sems[b]).wait()
                pltpu.sync_copy(bufs[b], acc.at[dst_v.at[j]], add=True)
                pltpu.async_copy(x_hbm.at[src_v.at[j + nbuf]], bufs[b], sems[b])
            return carry

        lax.fori_loop(0, C // nbuf - 1, group, 0)
        for b in range(nbuf):
            j = C - nbuf + b
            pltpu.make_async_copy(x_hbm.at[src_v.at[j]], bufs[b], sems[b]).wait()
            pltpu.sync_copy(bufs[b], acc.at[dst_v.at[j]], add=True)

        plsc.subcore_barrier()
        pltpu.sync_copy(acc.at[pl.ds(s * STRIPE, STRIPE)],
                        out_hbm.at[pl.ds(c * N_PAD + s * STRIPE, STRIPE)])

    return pl.kernel(
        body,
        out_type=jax.ShapeDtypeStruct((2 * N_PAD, d), jnp.float32),
        mesh=mesh,
        compiler_params=pltpu.CompilerParams(use_tc_tiling_on_sc=False),
        scratch_types=[
            pltpu.VMEM((C, K), jnp.int32),
            pltpu.VMEM((C, K), jnp.int32),
            [pltpu.VMEM((K, d), jnp.float32) for _ in range(nbuf)],
            [pltpu.SemaphoreType.DMA for _ in range(nbuf)],
            pltpu.VMEM_SHARED((ACC_ROWS, d), jnp.float32),
        ],
    )


_agg_aug = _make_agg(D_AUG, 2)
_agg_plain = _make_agg(D, 3)


def _dense1_body(p0, p1, x_ref, wl, wr, b, h_ref, inv_ref):
    s = p0[...] + p1[...]
    deg = s[:, D:D + 1]
    inv = 1.0 / jnp.maximum(deg, 1.0)
    mean = s[:, :D] * inv
    h = (jnp.dot(mean, wl[...], preferred_element_type=jnp.float32)
         + jnp.dot(x_ref[...], wr[...], preferred_element_type=jnp.float32)
         + b[...])
    h_ref[...] = jnp.maximum(h, 0.0)
    inv_ref[...] = inv


_dense1 = pl.pallas_call(
    _dense1_body,
    grid=(N_PAD // BLK,),
    in_specs=[
        pl.BlockSpec((BLK, D_AUG), lambda i: (i, 0)),
        pl.BlockSpec((BLK, D_AUG), lambda i: (i, 0)),
        pl.BlockSpec((BLK, D), lambda i: (i, 0)),
        pl.BlockSpec((D, D), lambda i: (0, 0)),
        pl.BlockSpec((D, D), lambda i: (0, 0)),
        pl.BlockSpec((1, D), lambda i: (0, 0)),
    ],
    out_specs=[pl.BlockSpec((BLK, D), lambda i: (i, 0)),
               pl.BlockSpec((BLK, 1), lambda i: (i, 0))],
    out_shape=[jax.ShapeDtypeStruct((N_PAD, D), jnp.float32),
               jax.ShapeDtypeStruct((N_PAD, 1), jnp.float32)],
)


def _dense2_body(p0, p1, h_ref, inv_ref, wl, wr, b, out_ref):
    mean = (p0[...] + p1[...]) * inv_ref[...]
    out_ref[...] = (jnp.dot(mean, wl[...], preferred_element_type=jnp.float32)
                    + jnp.dot(h_ref[...], wr[...], preferred_element_type=jnp.float32)
                    + b[...])


_dense2 = pl.pallas_call(
    _dense2_body,
    grid=(N_PAD // BLK,),
    in_specs=[
        pl.BlockSpec((BLK, D), lambda i: (i, 0)),
        pl.BlockSpec((BLK, D), lambda i: (i, 0)),
        pl.BlockSpec((BLK, D), lambda i: (i, 0)),
        pl.BlockSpec((BLK, 1), lambda i: (i, 0)),
        pl.BlockSpec((D, D), lambda i: (0, 0)),
        pl.BlockSpec((D, D), lambda i: (0, 0)),
        pl.BlockSpec((1, D), lambda i: (0, 0)),
    ],
    out_specs=pl.BlockSpec((BLK, D), lambda i: (i, 0)),
    out_shape=jax.ShapeDtypeStruct((N_PAD, D), jnp.float32),
)


def kernel(x, edge_index, W1l, b1l, W1r, W2l, b2l, W2r):
    src = edge_index[0].astype(jnp.int32)
    dst = edge_index[1].astype(jnp.int32)
    src_t = jnp.concatenate([src, jnp.zeros((E_PAD - E,), jnp.int32)]).reshape(NW, C, K)
    dst_t = jnp.concatenate([dst, jnp.full((E_PAD - E,), N, jnp.int32)]).reshape(NW, C, K)

    x_aug = jnp.zeros((N_PAD, D_AUG), jnp.float32)
    x_aug = x_aug.at[:N, :D].set(x)
    x_aug = x_aug.at[:N, D].set(1.0)
    x_pad = x_aug[:, :D]
    z_aug = jnp.zeros((STRIPE, D_AUG), jnp.float32)
    z_plain = jnp.zeros((STRIPE, D), jnp.float32)

    p = _agg_aug(x_aug, src_t, dst_t, z_aug)
    h, inv = _dense1(p[:N_PAD], p[N_PAD:], x_pad, W1l.T, W1r.T, b1l[None, :])
    p2 = _agg_plain(h, src_t, dst_t, z_plain)
    out = _dense2(p2[:N_PAD], p2[N_PAD:], h, inv, W2l.T, W2r.T, b2l[None, :])
    return out[:N]


# speedup vs baseline: 1.0260x; 1.0260x over previous
import jax
import jax.numpy as jnp
from jax import lax
from jax.experimental import pallas as pl
from jax.experimental.pallas import tpu as pltpu
from jax.experimental.pallas import tpu_sc as plsc

N = 10000
E = 320000
D = 128
N_PAD = 10240
ACC_ROWS = 10000
D_AUG = 144
NW = 32
K = 64
C = 6 * (-(-E // (NW * K * 6)))
E_PAD = NW * C * K
STRIPE = ACC_ROWS // 16
BLK = 512


def _make_agg(d, nbuf):
    mesh = plsc.VectorSubcoreMesh(core_axis_name="c", subcore_axis_name="s")

    def body(x_hbm, src_hbm, dst_hbm, z_hbm, out_hbm, src_v, dst_v,
             bufs, sems, acc):
        c = lax.axis_index("c")
        s = lax.axis_index("s")
        wid = c * 16 + s
        pltpu.sync_copy(src_hbm.at[wid], src_v)
        pltpu.sync_copy(dst_hbm.at[wid], dst_v)
        pltpu.sync_copy(z_hbm, acc.at[pl.ds(s * STRIPE, STRIPE)])
        plsc.subcore_barrier()

        for b in range(nbuf):
            pltpu.async_copy(x_hbm.at[src_v.at[b]], bufs[b], sems[b])

        def group(g, carry):
            for b in range(nbuf):
                j = g * nbuf + b
                pltpu.make_async_copy(x_hbm.at[src_v.at[j]], bufs[b],
                                      sems[b]).wait()
                pltpu.sync_copy(bufs[b], acc.at[dst_v.at[j]], add=True)
                pltpu.async_copy(x_hbm.at[src_v.at[j + nbuf]], bufs[b], sems[b])
            return carry

        lax.fori_loop(0, C // nbuf - 1, group, 0)
        for b in range(nbuf):
            j = C - nbuf + b
            pltpu.make_async_copy(x_hbm.at[src_v.at[j]], bufs[b], sems[b]).wait()
            pltpu.sync_copy(bufs[b], acc.at[dst_v.at[j]], add=True)

        plsc.subcore_barrier()
        pltpu.sync_copy(acc.at[pl.ds(s * STRIPE, STRIPE)],
                        out_hbm.at[pl.ds(c * N_PAD + s * STRIPE, STRIPE)])

    return pl.kernel(
        body,
        out_type=jax.ShapeDtypeStruct((2 * N_PAD, d), jnp.float32),
        mesh=mesh,
        compiler_params=pltpu.CompilerParams(use_tc_tiling_on_sc=False),
        scratch_types=[
            pltpu.VMEM((C, K), jnp.int32),
            pltpu.VMEM((C, K), jnp.int32),
            [pltpu.VMEM((K, d), jnp.float32) for _ in range(nbuf)],
            [pltpu.SemaphoreType.DMA for _ in range(nbuf)],
            pltpu.VMEM_SHARED((ACC_ROWS, d), jnp.float32),
        ],
    )


_agg_aug = _make_agg(D_AUG, 2)
_agg_plain = _make_agg(D, 3)


def _dense1_body(p0, p1, x_ref, wl, wr, b, h_ref, inv_ref):
    s = p0[...] + p1[...]
    deg = s[:, D:D + 1]
    inv = 1.0 / jnp.maximum(deg, 1.0)
    mean = s[:, :D] * inv
    h = (jnp.dot(mean, wl[...], preferred_element_type=jnp.float32)
         + jnp.dot(x_ref[...], wr[...], preferred_element_type=jnp.float32)
         + b[...])
    row = pl.program_id(0) * BLK + lax.broadcasted_iota(jnp.int32, (BLK, 1), 0)
    h_ref[...] = jnp.where(row < N, jnp.maximum(h, 0.0), 0.0)
    inv_ref[...] = inv


_dense1 = pl.pallas_call(
    _dense1_body,
    grid=(N_PAD // BLK,),
    in_specs=[
        pl.BlockSpec((BLK, D_AUG), lambda i: (i, 0)),
        pl.BlockSpec((BLK, D_AUG), lambda i: (i, 0)),
        pl.BlockSpec((BLK, D), lambda i: (i, 0)),
        pl.BlockSpec((D, D), lambda i: (0, 0)),
        pl.BlockSpec((D, D), lambda i: (0, 0)),
        pl.BlockSpec((1, D), lambda i: (0, 0)),
    ],
    out_specs=[pl.BlockSpec((BLK, D), lambda i: (i, 0)),
               pl.BlockSpec((BLK, 1), lambda i: (i, 0))],
    out_shape=[jax.ShapeDtypeStruct((N_PAD, D), jnp.float32),
               jax.ShapeDtypeStruct((N_PAD, 1), jnp.float32)],
)


def _dense2_body(p0, p1, h_ref, inv_ref, wl, wr, b, out_ref):
    mean = (p0[...] + p1[...]) * inv_ref[...]
    out_ref[...] = (jnp.dot(mean, wl[...], preferred_element_type=jnp.float32)
                    + jnp.dot(h_ref[...], wr[...], preferred_element_type=jnp.float32)
                    + b[...])


_dense2 = pl.pallas_call(
    _dense2_body,
    grid=(N_PAD // BLK,),
    in_specs=[
        pl.BlockSpec((BLK, D), lambda i: (i, 0)),
        pl.BlockSpec((BLK, D), lambda i: (i, 0)),
        pl.BlockSpec((BLK, D), lambda i: (i, 0)),
        pl.BlockSpec((BLK, 1), lambda i: (i, 0)),
        pl.BlockSpec((D, D), lambda i: (0, 0)),
        pl.BlockSpec((D, D), lambda i: (0, 0)),
        pl.BlockSpec((1, D), lambda i: (0, 0)),
    ],
    out_specs=pl.BlockSpec((BLK, D), lambda i: (i, 0)),
    out_shape=jax.ShapeDtypeStruct((N_PAD, D), jnp.float32),
)


def kernel(x, edge_index, W1l, b1l, W1r, W2l, b2l, W2r):
    src = edge_index[0].astype(jnp.int32)
    dst = edge_index[1].astype(jnp.int32)
    pad_dst = jnp.arange(E_PAD - E, dtype=jnp.int32) % N
    src_t = jnp.concatenate([src, jnp.full((E_PAD - E,), N, jnp.int32)]).reshape(NW, C, K)
    dst_t = jnp.concatenate([dst, pad_dst]).reshape(NW, C, K)

    x_aug = jnp.zeros((N_PAD, D_AUG), jnp.float32)
    x_aug = x_aug.at[:N, :D].set(x)
    x_aug = x_aug.at[:N, D].set(1.0)
    x_pad = x_aug[:, :D]
    z_aug = jnp.zeros((STRIPE, D_AUG), jnp.float32)
    z_plain = jnp.zeros((STRIPE, D), jnp.float32)

    p = _agg_aug(x_aug, src_t, dst_t, z_aug)
    h, inv = _dense1(p[:N_PAD], p[N_PAD:], x_pad, W1l.T, W1r.T, b1l[None, :])
    p2 = _agg_plain(h, src_t, dst_t, z_plain)
    out = _dense2(p2[:N_PAD], p2[N_PAD:], h, inv, W2l.T, W2r.T, b2l[None, :])
    return out[:N]


# speedup vs baseline: 1.1541x; 1.1248x over previous
import jax
import jax.numpy as jnp
from jax import lax
from jax.experimental import pallas as pl
from jax.experimental.pallas import tpu as pltpu
from jax.experimental.pallas import tpu_sc as plsc

N = 10000
E = 320000
D = 128
N_PAD = 10240
ACC_ROWS = 10000
D_AUG = 144
NW = 32
KA = 128
IBA = 2
CBA = 56
EA = 16 * CBA * IBA * KA
KB = 256
IBB = 2
CBB = 12
EB_CAP = 16 * CBB * IBB * KB
E_PAD = EA + EB_CAP
STRIPE = ACC_ROWS // 16
BLK = 512


def _make_agg(d):
    mesh = plsc.VectorSubcoreMesh(core_axis_name="c", subcore_axis_name="s")

    def body(x_hbm, idxa_hbm, idxb_hbm, z_hbm, out_hbm,
             iba, ibb, buf, sem_g, sem_i, acc):
        c = lax.axis_index("c")
        s = lax.axis_index("s")
        pltpu.sync_copy(z_hbm, acc.at[pl.ds(s * STRIPE, STRIPE)])
        plsc.subcore_barrier()

        bufs = [buf.at[pl.ds(0, KA)], buf.at[pl.ds(KA, KA)]]

        @pl.when(c == 0)
        def _sc0():
            pltpu.sync_copy(idxa_hbm.at[s, 0], iba[0])
            pltpu.async_copy(idxa_hbm.at[s, 1], iba[1], sem_i[1])
            pltpu.async_copy(x_hbm.at[iba[0].at[0, 0]], bufs[0], sem_g[0])
            pltpu.async_copy(x_hbm.at[iba[0].at[0, 1]], bufs[1], sem_g[1])

            def group(g, carry):
                for pp in range(2):
                    m = 2 * g + pp
                    pltpu.make_async_copy(idxa_hbm.at[s, 0], iba[1 - pp],
                                          sem_i[1 - pp]).wait()
                    for b in range(2):
                        pltpu.make_async_copy(x_hbm.at[iba[pp].at[0, b]],
                                              bufs[b], sem_g[b]).wait()
                        pltpu.sync_copy(bufs[b], acc.at[iba[pp].at[1, b]],
                                        add=True)
                        pltpu.async_copy(x_hbm.at[iba[1 - pp].at[0, b]],
                                         bufs[b], sem_g[b])
                    pltpu.async_copy(idxa_hbm.at[s, m + 2], iba[pp], sem_i[pp])
                return carry

            lax.fori_loop(0, CBA // 2, group, 0)
            pltpu.make_async_copy(x_hbm.at[iba[0].at[0, 0]], bufs[0],
                                  sem_g[0]).wait()
            pltpu.make_async_copy(x_hbm.at[iba[0].at[0, 1]], bufs[1],
                                  sem_g[1]).wait()
            pltpu.make_async_copy(idxa_hbm.at[s, 0], iba[1], sem_i[1]).wait()

        @pl.when(c == 1)
        def _sc1():
            pltpu.sync_copy(idxb_hbm.at[s, 0], ibb[0])
            pltpu.async_copy(idxb_hbm.at[s, 1], ibb[1], sem_i[1])

            def group(g, carry):
                for pp in range(2):
                    m = 2 * g + pp
                    for b in range(2):
                        pltpu.sync_copy(x_hbm.at[ibb[pp].at[0, b]], buf)
                        pltpu.sync_copy(buf, acc.at[ibb[pp].at[1, b]], add=True)
                    pltpu.make_async_copy(idxb_hbm.at[s, 0], ibb[1 - pp],
                                          sem_i[1 - pp]).wait()
                    pltpu.async_copy(idxb_hbm.at[s, m + 2], ibb[pp], sem_i[pp])
                return carry

            lax.fori_loop(0, CBB // 2, group, 0)
            pltpu.make_async_copy(idxb_hbm.at[s, 0], ibb[1], sem_i[1]).wait()

        plsc.subcore_barrier()
        pltpu.sync_copy(acc.at[pl.ds(s * STRIPE, STRIPE)],
                        out_hbm.at[pl.ds(c * N_PAD + s * STRIPE, STRIPE)])

    return pl.kernel(
        body,
        out_type=jax.ShapeDtypeStruct((2 * N_PAD, d), jnp.float32),
        mesh=mesh,
        compiler_params=pltpu.CompilerParams(use_tc_tiling_on_sc=False),
        scratch_types=[
            [pltpu.VMEM((2, IBA, KA), jnp.int32) for _ in range(2)],
            [pltpu.VMEM((2, IBB, KB), jnp.int32) for _ in range(2)],
            pltpu.VMEM((KB, d), jnp.float32),
            [pltpu.SemaphoreType.DMA for _ in range(2)],
            [pltpu.SemaphoreType.DMA for _ in range(2)],
            pltpu.VMEM_SHARED((ACC_ROWS, d), jnp.float32),
        ],
    )


_agg_aug = _make_agg(D_AUG)
_agg_plain = _make_agg(D)


def _dense1_body(p0, p1, x_ref, wl, wr, b, h_ref, inv_ref):
    s = p0[...] + p1[...]
    deg = s[:, D:D + 1]
    inv = 1.0 / jnp.maximum(deg, 1.0)
    mean = s[:, :D] * inv
    h = (jnp.dot(mean, wl[...], preferred_element_type=jnp.float32)
         + jnp.dot(x_ref[...], wr[...], preferred_element_type=jnp.float32)
         + b[...])
    row = pl.program_id(0) * BLK + lax.broadcasted_iota(jnp.int32, (BLK, 1), 0)
    h_ref[...] = jnp.where(row < N, jnp.maximum(h, 0.0), 0.0)
    inv_ref[...] = inv


_dense1 = pl.pallas_call(
    _dense1_body,
    grid=(N_PAD // BLK,),
    in_specs=[
        pl.BlockSpec((BLK, D_AUG), lambda i: (i, 0)),
        pl.BlockSpec((BLK, D_AUG), lambda i: (i, 0)),
        pl.BlockSpec((BLK, D), lambda i: (i, 0)),
        pl.BlockSpec((D, D), lambda i: (0, 0)),
        pl.BlockSpec((D, D), lambda i: (0, 0)),
        pl.BlockSpec((1, D), lambda i: (0, 0)),
    ],
    out_specs=[pl.BlockSpec((BLK, D), lambda i: (i, 0)),
               pl.BlockSpec((BLK, 1), lambda i: (i, 0))],
    out_shape=[jax.ShapeDtypeStruct((N_PAD, D), jnp.float32),
               jax.ShapeDtypeStruct((N_PAD, 1), jnp.float32)],
)


def _dense2_body(p0, p1, h_ref, inv_ref, wl, wr, b, out_ref):
    mean = (p0[...] + p1[...]) * inv_ref[...]
    out_ref[...] = (jnp.dot(mean, wl[...], preferred_element_type=jnp.float32)
                    + jnp.dot(h_ref[...], wr[...], preferred_element_type=jnp.float32)
                    + b[...])


_dense2 = pl.pallas_call(
    _dense2_body,
    grid=(N_PAD // BLK,),
    in_specs=[
        pl.BlockSpec((BLK, D), lambda i: (i, 0)),
        pl.BlockSpec((BLK, D), lambda i: (i, 0)),
        pl.BlockSpec((BLK, D), lambda i: (i, 0)),
        pl.BlockSpec((BLK, 1), lambda i: (i, 0)),
        pl.BlockSpec((D, D), lambda i: (0, 0)),
        pl.BlockSpec((D, D), lambda i: (0, 0)),
        pl.BlockSpec((1, D), lambda i: (0, 0)),
    ],
    out_specs=pl.BlockSpec((BLK, D), lambda i: (i, 0)),
    out_shape=jax.ShapeDtypeStruct((N_PAD, D), jnp.float32),
)


def kernel(x, edge_index, W1l, b1l, W1r, W2l, b2l, W2r):
    src = edge_index[0].astype(jnp.int32)
    dst = edge_index[1].astype(jnp.int32)
    pad_dst = jnp.arange(E_PAD - E, dtype=jnp.int32) % N
    src_p = jnp.concatenate([src, jnp.full((E_PAD - E,), N, jnp.int32)])
    dst_p = jnp.concatenate([dst, pad_dst])
    sa = src_p[:EA].reshape(16, CBA, IBA, KA)
    da = dst_p[:EA].reshape(16, CBA, IBA, KA)
    idxa = jnp.stack([sa, da], axis=2)
    dummy_a = jnp.stack(
        [jnp.full((16, 2, IBA, KA), N, jnp.int32),
         jnp.zeros((16, 2, IBA, KA), jnp.int32)], axis=2)
    idxa = jnp.concatenate([idxa, dummy_a], axis=1)
    sb = src_p[EA:].reshape(16, CBB, IBB, KB)
    db = dst_p[EA:].reshape(16, CBB, IBB, KB)
    idxb = jnp.stack([sb, db], axis=2)
    dummy_b = jnp.stack(
        [jnp.full((16, 2, IBB, KB), N, jnp.int32),
         jnp.zeros((16, 2, IBB, KB), jnp.int32)], axis=2)
    idxb = jnp.concatenate([idxb, dummy_b], axis=1)

    x_aug = jnp.zeros((N_PAD, D_AUG), jnp.float32)
    x_aug = x_aug.at[:N, :D].set(x)
    x_aug = x_aug.at[:N, D].set(1.0)
    x_pad = x_aug[:, :D]
    z_aug = jnp.zeros((STRIPE, D_AUG), jnp.float32)
    z_plain = jnp.zeros((STRIPE, D), jnp.float32)

    p = _agg_aug(x_aug, idxa, idxb, z_aug)
    h, inv = _dense1(p[:N_PAD], p[N_PAD:], x_pad, W1l.T, W1r.T, b1l[None, :])
    p2 = _agg_plain(h, idxa, idxb, z_plain)
    out = _dense2(p2[:N_PAD], p2[N_PAD:], h, inv, W2l.T, W2r.T, b2l[None, :])
    return out[:N]


# speedup vs baseline: 2.0042x; 1.7366x over previous
import jax
import jax.numpy as jnp
from jax import lax
from jax.experimental import pallas as pl
from jax.experimental.pallas import tpu as pltpu
from jax.experimental.pallas import tpu_sc as plsc

N = 10000
E = 320000
D = 128
N_PAD = 10240
ACC_ROWS = 10000
D_AUG = 144
NW = 32
KA = 64
SEGA = 108
KB = 128
SEGB = 25
EA = 16 * 2 * SEGA * KA
EB_CAP = 16 * 2 * SEGB * KB
E_PAD = EA + EB_CAP
STRIPE = ACC_ROWS // 16
BLK = 512


def _make_agg(d, nbuf):
    mesh = plsc.VectorSubcoreMesh(core_axis_name="c", subcore_axis_name="s")

    def body(x_hbm, idxa_hbm, idxb_hbm, z_hbm, out_hbm,
             srcva, dstva, srcvb, dstvb, buf, sems, acc):
        c = lax.axis_index("c")
        s = lax.axis_index("s")
        pltpu.sync_copy(z_hbm, acc.at[pl.ds(s * STRIPE, STRIPE)])
        plsc.subcore_barrier()

        bufs = [buf.at[pl.ds(b * KA, KA)] for b in range(nbuf)]

        @pl.when(c == 0)
        def _sc0():
            for seg in range(2):
                pltpu.sync_copy(idxa_hbm.at[s, seg, 0], srcva)
                pltpu.sync_copy(idxa_hbm.at[s, seg, 1], dstva)
                for b in range(nbuf):
                    pltpu.async_copy(x_hbm.at[srcva.at[b]], bufs[b], sems[b])

                def group(g, carry):
                    for b in range(nbuf):
                        j = g * nbuf + b
                        pltpu.make_async_copy(x_hbm.at[srcva.at[j]], bufs[b],
                                              sems[b]).wait()
                        pltpu.sync_copy(bufs[b], acc.at[dstva.at[j]], add=True)
                        pltpu.async_copy(x_hbm.at[srcva.at[j + nbuf]],
                                         bufs[b], sems[b])
                    return carry

                lax.fori_loop(0, SEGA // nbuf - 1, group, 0)
                for b in range(nbuf):
                    j = SEGA - nbuf + b
                    pltpu.make_async_copy(x_hbm.at[srcva.at[j]], bufs[b],
                                          sems[b]).wait()
                    pltpu.sync_copy(bufs[b], acc.at[dstva.at[j]], add=True)

        @pl.when(c == 1)
        def _sc1():
            sbuf = buf.at[pl.ds(0, KB)]
            for seg in range(2):
                pltpu.sync_copy(idxb_hbm.at[s, seg, 0], srcvb)
                pltpu.sync_copy(idxb_hbm.at[s, seg, 1], dstvb)

                def step(j, carry):
                    pltpu.sync_copy(x_hbm.at[srcvb.at[j]], sbuf)
                    pltpu.sync_copy(sbuf, acc.at[dstvb.at[j]], add=True)
                    return carry

                lax.fori_loop(0, SEGB, step, 0)

        plsc.subcore_barrier()
        pltpu.sync_copy(acc.at[pl.ds(s * STRIPE, STRIPE)],
                        out_hbm.at[pl.ds(c * N_PAD + s * STRIPE, STRIPE)])

    return pl.kernel(
        body,
        out_type=jax.ShapeDtypeStruct((2 * N_PAD, d), jnp.float32),
        mesh=mesh,
        compiler_params=pltpu.CompilerParams(use_tc_tiling_on_sc=False),
        scratch_types=[
            pltpu.VMEM((SEGA, KA), jnp.int32),
            pltpu.VMEM((SEGA, KA), jnp.int32),
            pltpu.VMEM((SEGB, KB), jnp.int32),
            pltpu.VMEM((SEGB, KB), jnp.int32),
            pltpu.VMEM((nbuf * KA if nbuf * KA > KB else KB, d), jnp.float32),
            [pltpu.SemaphoreType.DMA for _ in range(nbuf)],
            pltpu.VMEM_SHARED((ACC_ROWS, d), jnp.float32),
        ],
    )


_agg_aug = _make_agg(D_AUG, 2)
_agg_plain = _make_agg(D, 3)


def _dense1_body(p0, p1, x_ref, wl, wr, b, h_ref, inv_ref):
    s = p0[...] + p1[...]
    deg = s[:, D:D + 1]
    inv = 1.0 / jnp.maximum(deg, 1.0)
    mean = s[:, :D] * inv
    h = (jnp.dot(mean, wl[...], preferred_element_type=jnp.float32)
         + jnp.dot(x_ref[...], wr[...], preferred_element_type=jnp.float32)
         + b[...])
    row = pl.program_id(0) * BLK + lax.broadcasted_iota(jnp.int32, (BLK, 1), 0)
    h_ref[...] = jnp.where(row < N, jnp.maximum(h, 0.0), 0.0)
    inv_ref[...] = inv


_dense1 = pl.pallas_call(
    _dense1_body,
    grid=(N_PAD // BLK,),
    in_specs=[
        pl.BlockSpec((BLK, D_AUG), lambda i: (i, 0)),
        pl.BlockSpec((BLK, D_AUG), lambda i: (i, 0)),
        pl.BlockSpec((BLK, D), lambda i: (i, 0)),
        pl.BlockSpec((D, D), lambda i: (0, 0)),
        pl.BlockSpec((D, D), lambda i: (0, 0)),
        pl.BlockSpec((1, D), lambda i: (0, 0)),
    ],
    out_specs=[pl.BlockSpec((BLK, D), lambda i: (i, 0)),
               pl.BlockSpec((BLK, 1), lambda i: (i, 0))],
    out_shape=[jax.ShapeDtypeStruct((N_PAD, D), jnp.float32),
               jax.ShapeDtypeStruct((N_PAD, 1), jnp.float32)],
)


def _dense2_body(p0, p1, h_ref, inv_ref, wl, wr, b, out_ref):
    mean = (p0[...] + p1[...]) * inv_ref[...]
    out_ref[...] = (jnp.dot(mean, wl[...], preferred_element_type=jnp.float32)
                    + jnp.dot(h_ref[...], wr[...], preferred_element_type=jnp.float32)
                    + b[...])


_dense2 = pl.pallas_call(
    _dense2_body,
    grid=(N_PAD // BLK,),
    in_specs=[
        pl.BlockSpec((BLK, D), lambda i: (i, 0)),
        pl.BlockSpec((BLK, D), lambda i: (i, 0)),
        pl.BlockSpec((BLK, D), lambda i: (i, 0)),
        pl.BlockSpec((BLK, 1), lambda i: (i, 0)),
        pl.BlockSpec((D, D), lambda i: (0, 0)),
        pl.BlockSpec((D, D), lambda i: (0, 0)),
        pl.BlockSpec((1, D), lambda i: (0, 0)),
    ],
    out_specs=pl.BlockSpec((BLK, D), lambda i: (i, 0)),
    out_shape=jax.ShapeDtypeStruct((N_PAD, D), jnp.float32),
)


def kernel(x, edge_index, W1l, b1l, W1r, W2l, b2l, W2r):
    src = edge_index[0].astype(jnp.int32)
    dst = edge_index[1].astype(jnp.int32)
    pad_dst = jnp.arange(E_PAD - E, dtype=jnp.int32) % N
    src_p = jnp.concatenate([src, jnp.full((E_PAD - E,), N, jnp.int32)])
    dst_p = jnp.concatenate([dst, pad_dst])
    sa = src_p[:EA].reshape(16, 2, SEGA, KA)
    da = dst_p[:EA].reshape(16, 2, SEGA, KA)
    idxa = jnp.stack([sa, da], axis=2)
    sb = src_p[EA:].reshape(16, 2, SEGB, KB)
    db = dst_p[EA:].reshape(16, 2, SEGB, KB)
    idxb = jnp.stack([sb, db], axis=2)

    x_aug = jnp.zeros((N_PAD, D_AUG), jnp.float32)
    x_aug = x_aug.at[:N, :D].set(x)
    x_aug = x_aug.at[:N, D].set(1.0)
    x_pad = x_aug[:, :D]
    z_aug = jnp.zeros((STRIPE, D_AUG), jnp.float32)
    z_plain = jnp.zeros((STRIPE, D), jnp.float32)

    p = _agg_aug(x_aug, idxa, idxb, z_aug)
    h, inv = _dense1(p[:N_PAD], p[N_PAD:], x_pad, W1l.T, W1r.T, b1l[None, :])
    p2 = _agg_plain(h, idxa, idxb, z_plain)
    out = _dense2(p2[:N_PAD], p2[N_PAD:], h, inv, W2l.T, W2r.T, b2l[None, :])
    return out[:N]
